# baseline (speedup 1.0000x reference)
import functools
import math

import jax
import jax.numpy as jnp
from jax import lax
from jax.experimental import pallas as pl
from jax.experimental.pallas import tpu as pltpu
from jax.experimental.pallas import tpu_sc as plsc

D_MODEL = 64
SCALE = math.sqrt(D_MODEL)
LANES = 16
CHUNK = 1024
UNROLL = 8


def _make_kernel(n_idx):
    info = plsc.get_sparse_core_info()
    nc, ns = info.num_cores, info.num_subcores
    nw = nc * ns
    assert n_idx % (nw * CHUNK) == 0
    b_per_w = n_idx // nw
    n_chunks = b_per_w // CHUNK
    vregs_per_row = D_MODEL // LANES

    mesh = plsc.VectorSubcoreMesh(core_axis_name="c", subcore_axis_name="s")

    @functools.partial(
        pl.kernel,
        mesh=mesh,
        out_type=jax.ShapeDtypeStruct((n_idx, D_MODEL), jnp.float32),
        scratch_types=[
            pltpu.VMEM((CHUNK,), jnp.int32),
            pltpu.VMEM((CHUNK, D_MODEL), jnp.float32),
            pltpu.SemaphoreType.DMA,
        ],
        compiler_params=pltpu.CompilerParams(use_tc_tiling_on_sc=False),
    )
    def body(idx_hbm, table_hbm, out_hbm, idx_v, rows_v, sem):
        wid = lax.axis_index("s") * nc + lax.axis_index("c")
        base = wid * b_per_w

        def one_chunk(c, _):
            start = base + c * CHUNK
            pltpu.sync_copy(idx_hbm.at[pl.ds(start, CHUNK)], idx_v)
            pltpu.async_copy(table_hbm.at[idx_v], rows_v, sem).wait()

            def scale_body(i, _):
                rb = i * UNROLL
                for u in range(UNROLL):
                    for k in range(vregs_per_row):
                        sl = pl.ds(k * LANES, LANES)
                        rows_v[rb + u, sl] = rows_v[rb + u, sl] * SCALE
                return 0

            lax.fori_loop(0, CHUNK // UNROLL, scale_body, 0)
            pltpu.sync_copy(rows_v, out_hbm.at[pl.ds(start, CHUNK)])
            return 0

        lax.fori_loop(0, n_chunks, one_chunk, 0)

    return body


def kernel(x, table):
    b, s = x.shape
    idx = x.reshape(-1).astype(jnp.int32)
    out = _make_kernel(idx.shape[0])(idx, table)
    return out.reshape(b, s, D_MODEL)


# speedup vs baseline: 1.0639x; 1.0639x over previous
import functools
import math

import jax
import jax.numpy as jnp
from jax import lax
from jax.experimental import pallas as pl
from jax.experimental.pallas import tpu as pltpu
from jax.experimental.pallas import tpu_sc as plsc

D_MODEL = 64
SCALE = math.sqrt(D_MODEL)
LANES = 16
CHUNK = 800
UNROLL = 8


def _make_kernel(n_idx):
    info = plsc.get_sparse_core_info()
    nc, ns = info.num_cores, info.num_subcores
    nw = nc * ns
    assert n_idx % (nw * CHUNK) == 0
    b_per_w = n_idx // nw
    n_chunks = b_per_w // CHUNK
    assert n_chunks % 2 == 0
    n_pairs = n_chunks // 2
    vregs_per_row = D_MODEL // LANES

    mesh = plsc.VectorSubcoreMesh(core_axis_name="c", subcore_axis_name="s")

    @functools.partial(
        pl.kernel,
        mesh=mesh,
        out_type=jax.ShapeDtypeStruct((n_idx, D_MODEL), jnp.float32),
        scratch_types=[
            pltpu.VMEM((b_per_w,), jnp.int32),
            pltpu.VMEM((CHUNK, D_MODEL), jnp.float32),
            pltpu.VMEM((CHUNK, D_MODEL), jnp.float32),
            pltpu.SemaphoreType.DMA,
            pltpu.SemaphoreType.DMA,
            pltpu.SemaphoreType.DMA,
            pltpu.SemaphoreType.DMA,
        ],
        compiler_params=pltpu.CompilerParams(use_tc_tiling_on_sc=False),
    )
    def body(idx_hbm, table_hbm, out_hbm, idx_all, rows0, rows1,
             gsem0, gsem1, osem0, osem1):
        wid = lax.axis_index("s") * nc + lax.axis_index("c")
        base = wid * b_per_w
        rows_v = (rows0, rows1)
        gsem = (gsem0, gsem1)
        osem = (osem0, osem1)

        pltpu.sync_copy(idx_hbm.at[pl.ds(base, b_per_w)], idx_all)

        def start_gather(c, b):
            pltpu.async_copy(table_hbm.at[idx_all.at[pl.ds(c * CHUNK, CHUNK)]],
                             rows_v[b], gsem[b])

        def wait_gather(c, b):
            pltpu.make_async_copy(
                table_hbm.at[idx_all.at[pl.ds(c * CHUNK, CHUNK)]],
                rows_v[b], gsem[b]).wait()

        def start_writeback(c, b):
            pltpu.async_copy(rows_v[b],
                             out_hbm.at[pl.ds(base + c * CHUNK, CHUNK)],
                             osem[b])

        def wait_writeback(c, b):
            pltpu.make_async_copy(rows_v[b],
                                  out_hbm.at[pl.ds(base + c * CHUNK, CHUNK)],
                                  osem[b]).wait()

        def scale(b):
            rows = rows_v[b]

            def scale_body(i, _):
                rb = i * UNROLL
                for u in range(UNROLL):
                    for k in range(vregs_per_row):
                        sl = pl.ds(k * LANES, LANES)
                        rows[rb + u, sl] = rows[rb + u, sl] * SCALE
                return 0

            lax.fori_loop(0, CHUNK // UNROLL, scale_body, 0)

        start_gather(0, 0)

        def pair_body(p, _):
            c0 = 2 * p
            c1 = c0 + 1
            wait_gather(c0, 0)

            @pl.when(p > 0)
            def _():
                wait_writeback(c0 - 1, 1)

            start_gather(c1, 1)
            scale(0)
            start_writeback(c0, 0)
            wait_gather(c1, 1)
            wait_writeback(c0, 0)

            @pl.when(p < n_pairs - 1)
            def _():
                start_gather(c0 + 2, 0)

            scale(1)
            start_writeback(c1, 1)
            return 0

        lax.fori_loop(0, n_pairs, pair_body, 0)
        wait_writeback(n_chunks - 1, 1)

    return body


def kernel(x, table):
    b, s = x.shape
    idx = x.reshape(-1).astype(jnp.int32)
    out = _make_kernel(idx.shape[0])(idx, table)
    return out.reshape(b, s, D_MODEL)


# speedup vs baseline: 1.0639x; 1.0000x over previous
import functools
import math

import jax
import jax.numpy as jnp
from jax import lax
from jax.experimental import pallas as pl
from jax.experimental.pallas import tpu as pltpu
from jax.experimental.pallas import tpu_sc as plsc

D_MODEL = 64
SCALE = math.sqrt(D_MODEL)
LANES = 16
SEQ_CHUNK = 4


def _make_kernel(n_seq, seq_len):
    info = plsc.get_sparse_core_info()
    nc, ns = info.num_cores, info.num_subcores
    nw = nc * ns
    assert n_seq % nw == 0
    seq_per_w = n_seq // nw
    assert seq_per_w % SEQ_CHUNK == 0
    n_chunks = seq_per_w // SEQ_CHUNK
    assert n_chunks % 2 == 0
    n_pairs = n_chunks // 2
    vregs_per_row = D_MODEL // LANES

    mesh = plsc.VectorSubcoreMesh(core_axis_name="c", subcore_axis_name="s")

    @functools.partial(
        pl.kernel,
        mesh=mesh,
        out_type=jax.ShapeDtypeStruct((n_seq, seq_len, D_MODEL), jnp.float32),
        scratch_types=[
            pltpu.VMEM((seq_per_w, seq_len), jnp.int32),
            pltpu.VMEM((SEQ_CHUNK, seq_len, D_MODEL), jnp.float32),
            pltpu.VMEM((SEQ_CHUNK, seq_len, D_MODEL), jnp.float32),
            pltpu.SemaphoreType.DMA,
            pltpu.SemaphoreType.DMA,
            pltpu.SemaphoreType.DMA,
            pltpu.SemaphoreType.DMA,
        ],
        compiler_params=pltpu.CompilerParams(use_tc_tiling_on_sc=False),
    )
    def body(x_hbm, table_hbm, out_hbm, idx_all, rows0, rows1,
             gsem0, gsem1, osem0, osem1):
        wid = lax.axis_index("s") * nc + lax.axis_index("c")
        base = wid * seq_per_w
        rows_v = (rows0, rows1)
        gsem = (gsem0, gsem1)
        osem = (osem0, osem1)

        pltpu.sync_copy(x_hbm.at[pl.ds(base, seq_per_w)], idx_all)

        def start_gather(c, b):
            for s in range(SEQ_CHUNK):
                pltpu.async_copy(
                    table_hbm.at[idx_all.at[c * SEQ_CHUNK + s]],
                    rows_v[b].at[s], gsem[b])

        def wait_gather(c, b):
            for s in range(SEQ_CHUNK):
                pltpu.make_async_copy(
                    table_hbm.at[idx_all.at[c * SEQ_CHUNK + s]],
                    rows_v[b].at[s], gsem[b]).wait()

        def start_writeback(c, b):
            pltpu.async_copy(
                rows_v[b],
                out_hbm.at[pl.ds(base + c * SEQ_CHUNK, SEQ_CHUNK)],
                osem[b])

        def wait_writeback(c, b):
            pltpu.make_async_copy(
                rows_v[b],
                out_hbm.at[pl.ds(base + c * SEQ_CHUNK, SEQ_CHUNK)],
                osem[b]).wait()

        def scale(b):
            rows = rows_v[b]

            def scale_body(r, _):
                for s in range(SEQ_CHUNK):
                    for k in range(vregs_per_row):
                        sl = pl.ds(k * LANES, LANES)
                        rows[s, r, sl] = rows[s, r, sl] * SCALE
                return 0

            lax.fori_loop(0, seq_len, scale_body, 0)

        start_gather(0, 0)

        def pair_body(p, _):
            c0 = 2 * p
            c1 = c0 + 1
            wait_gather(c0, 0)

            @pl.when(p > 0)
            def _():
                wait_writeback(c0 - 1, 1)

            start_gather(c1, 1)
            scale(0)
            start_writeback(c0, 0)
            wait_gather(c1, 1)
            wait_writeback(c0, 0)

            @pl.when(p < n_pairs - 1)
            def _():
                start_gather(c0 + 2, 0)

            scale(1)
            start_writeback(c1, 1)
            return 0

        lax.fori_loop(0, n_pairs, pair_body, 0)
        wait_writeback(n_chunks - 1, 1)

    return body


def kernel(x, table):
    n_seq, seq_len = x.shape
    return _make_kernel(n_seq, seq_len)(x.astype(jnp.int32), table)
